# Initial kernel scaffold; baseline (speedup 1.0000x reference)
import jax
import jax.numpy as jnp
from jax.experimental import pallas as pl


def kernel(vertices, colors, triangles):
    raise NotImplementedError("write your pallas kernel here")


# speedup vs baseline: 1.4280x; 1.4280x over previous
import functools

import jax
import jax.numpy as jnp
from jax import lax
from jax.experimental import pallas as pl
from jax.experimental.pallas import tpu as pltpu
from jax.experimental.pallas import tpu_sc as plsc

H = 112
W = 112
L = 16
BAND_ROWS = H // 16
BAND_PIX = BAND_ROWS * W
W_CHUNKS = W // L
NEG = -999999.0


def _iota():
    return lax.iota(jnp.int32, L)


def _sc_body(nv, nt, ng, verts_hbm, cols_hbm, tris_hbm, mask_out, img_out,
             vx, vy, vz, cr, cg, cb, t0, t1, t2,
             umin_r, umax_r, vmin_r, vmax_r, dep_r, txr, txg, txb,
             zbuf, idb, om, o0, o1, o2):
    c = lax.axis_index("c")
    s = lax.axis_index("s")
    band_lo = s * BAND_ROWS

    vbase = c * 3 * nv
    pltpu.sync_copy(verts_hbm.at[pl.ds(vbase, nv)], vx)
    pltpu.sync_copy(verts_hbm.at[pl.ds(vbase + nv, nv)], vy)
    pltpu.sync_copy(verts_hbm.at[pl.ds(vbase + 2 * nv, nv)], vz)
    pltpu.sync_copy(cols_hbm.at[pl.ds(vbase, nv)], cr)
    pltpu.sync_copy(cols_hbm.at[pl.ds(vbase + nv, nv)], cg)
    pltpu.sync_copy(cols_hbm.at[pl.ds(vbase + 2 * nv, nv)], cb)
    pltpu.sync_copy(tris_hbm.at[pl.ds(0, nt)], t0.at[pl.ds(0, nt)])
    pltpu.sync_copy(tris_hbm.at[pl.ds(nt, nt)], t1.at[pl.ds(0, nt)])
    pltpu.sync_copy(tris_hbm.at[pl.ds(2 * nt, nt)], t2.at[pl.ds(0, nt)])

    lastg = (ng - 1) * L
    tail_ok = (lastg + _iota()) < nt
    for tref in (t0, t1, t2):
        tv = tref[pl.ds(lastg, L)]
        tref[pl.ds(lastg, L)] = jnp.where(tail_ok, tv, 0)

    def meta_step(g, _):
        sl = pl.ds(g * L, L)
        i0 = t0[sl]
        i1 = t1[sl]
        i2 = t2[sl]
        dep_r[sl] = (plsc.load_gather(vz, [i0]) + plsc.load_gather(vz, [i1])
                     + plsc.load_gather(vz, [i2])) / 3.0
        x0 = plsc.load_gather(vx, [i0])
        x1 = plsc.load_gather(vx, [i1])
        x2 = plsc.load_gather(vx, [i2])
        xm = jnp.minimum(jnp.minimum(x0, x1), x2)
        xM = jnp.maximum(jnp.maximum(x0, x1), x2)
        y0 = plsc.load_gather(vy, [i0])
        y1 = plsc.load_gather(vy, [i1])
        y2 = plsc.load_gather(vy, [i2])
        ym = jnp.minimum(jnp.minimum(y0, y1), y2)
        yM = jnp.maximum(jnp.maximum(y0, y1), y2)
        xmi = xm.astype(jnp.int32)
        umin = xmi + jnp.where(xmi.astype(jnp.float32) < xm, 1, 0)
        umax = xM.astype(jnp.int32)
        ymi = ym.astype(jnp.int32)
        vmin = ymi + jnp.where(ymi.astype(jnp.float32) < ym, 1, 0)
        vmax = yM.astype(jnp.int32)
        umin = jnp.maximum(umin, 0)
        umax = jnp.minimum(umax, W - 1)
        vmin = jnp.maximum(vmin, 0)
        vmax = jnp.minimum(vmax, H - 1)
        pad = (g * L + _iota()) >= nt
        umin_r[sl] = jnp.where(pad, 1, umin)
        umax_r[sl] = jnp.where(pad, 0, umax)
        vmin_r[sl] = jnp.where(pad, 1, vmin)
        vmax_r[sl] = jnp.where(pad, 0, vmax)
        txr[sl] = (plsc.load_gather(cr, [i0]) + plsc.load_gather(cr, [i1])
                   + plsc.load_gather(cr, [i2])) / 3.0
        txg[sl] = (plsc.load_gather(cg, [i0]) + plsc.load_gather(cg, [i1])
                   + plsc.load_gather(cg, [i2])) / 3.0
        txb[sl] = (plsc.load_gather(cb, [i0]) + plsc.load_gather(cb, [i1])
                   + plsc.load_gather(cb, [i2])) / 3.0
        return _
    lax.fori_loop(0, ng, meta_step, 0, unroll=False)

    def init_step(p, _):
        sl = pl.ds(p * L, L)
        zbuf[sl] = jnp.full((L,), NEG, jnp.float32)
        idb[sl] = jnp.full((L,), -1, jnp.int32)
        return _
    lax.fori_loop(0, BAND_PIX // L, init_step, 0, unroll=False)

    def tri_group(g, _):
        base = g * L
        sl = pl.ds(base, L)
        umin_v = umin_r[sl]
        umax_v = umax_r[sl]
        vmin_v = vmin_r[sl]
        vmax_v = vmax_r[sl]
        dep_v = dep_r[sl]
        for l in range(L):
            u0 = umin_v[l]
            u1 = umax_v[l]
            rlo = jnp.maximum(vmin_v[l], band_lo)
            rhi = jnp.minimum(vmax_v[l], band_lo + (BAND_ROWS - 1))
            d = dep_v[l]
            t = base + l

            @pl.when(jnp.logical_and(u0 <= u1, rlo <= rhi))
            def _raster(u0=u0, u1=u1, rlo=rlo, rhi=rhi, d=d, t=t):
                c0 = lax.shift_right_logical(u0, 4)
                c1 = lax.shift_right_logical(u1, 4)

                def row_step(r, _r):
                    row_off = (r - band_lo) * W

                    def col_step(cc, _c):
                        off = row_off + cc * L
                        colv = cc * L + _iota()
                        inb = jnp.logical_and(colv >= u0, colv <= u1)
                        z = zbuf[pl.ds(off, L)]
                        win = jnp.logical_and(inb, d > z)
                        zbuf[pl.ds(off, L)] = jnp.where(win, d, z)
                        ids = idb[pl.ds(off, L)]
                        idb[pl.ds(off, L)] = jnp.where(win, t, ids)
                        return _c
                    lax.fori_loop(c0, c1 + 1, col_step, 0, unroll=False)
                    return _r
                lax.fori_loop(rlo, rhi + 1, row_step, 0, unroll=False)
        return _
    lax.fori_loop(0, ng, tri_group, 0, unroll=False)

    def out_step(p, _):
        sl = pl.ds(p * L, L)
        ids = idb[sl]
        hit = ids >= 0
        safe = jnp.maximum(ids, 0)
        om[sl] = jnp.where(hit, jnp.float32(1.0), jnp.float32(0.0))
        zero = jnp.float32(0.0)
        o0[sl] = jnp.where(hit, plsc.load_gather(txr, [safe]), zero)
        o1[sl] = jnp.where(hit, plsc.load_gather(txg, [safe]), zero)
        o2[sl] = jnp.where(hit, plsc.load_gather(txb, [safe]), zero)
        return _
    lax.fori_loop(0, BAND_PIX // L, out_step, 0, unroll=False)

    band = c * 16 + s
    pltpu.sync_copy(om, mask_out.at[pl.ds(band * BAND_PIX, BAND_PIX)])
    for ch, oref in enumerate((o0, o1, o2)):
        off = ((c * 3 + ch) * 16 + s) * BAND_PIX
        pltpu.sync_copy(oref, img_out.at[pl.ds(off, BAND_PIX)])


def kernel(vertices, colors, triangles):
    b, _, nv = vertices.shape
    nt = triangles.shape[1]
    ng = (nt + L - 1) // L
    ntp = ng * L

    mesh = plsc.VectorSubcoreMesh(core_axis_name="c", subcore_axis_name="s")
    run = pl.kernel(
        functools.partial(_sc_body, nv, nt, ng),
        out_type=(
            jax.ShapeDtypeStruct((b * 16 * BAND_PIX,), jnp.float32),
            jax.ShapeDtypeStruct((b * 3 * 16 * BAND_PIX,), jnp.float32),
        ),
        mesh=mesh,
        compiler_params=pltpu.CompilerParams(needs_layout_passes=False),
        scratch_types=(
            [pltpu.VMEM((nv,), jnp.float32) for _ in range(6)]
            + [pltpu.VMEM((ntp,), jnp.int32) for _ in range(3)]
            + [pltpu.VMEM((ntp,), jnp.int32) for _ in range(4)]
            + [pltpu.VMEM((ntp,), jnp.float32) for _ in range(4)]
            + [pltpu.VMEM((BAND_PIX,), jnp.float32),
               pltpu.VMEM((BAND_PIX,), jnp.int32)]
            + [pltpu.VMEM((BAND_PIX,), jnp.float32) for _ in range(4)]
        ),
    )
    mask_flat, img_flat = run(vertices.reshape(-1), colors.reshape(-1),
                              triangles.reshape(-1))
    face_mask = mask_flat.reshape(b, 1, H, W)
    new_image = img_flat.reshape(b, 3, H, W)
    return (face_mask, new_image)
